# baseline (speedup 1.0000x reference)
import functools

import jax
import jax.numpy as jnp
from jax import lax
from jax.experimental import pallas as pl
from jax.experimental.pallas import tpu as pltpu
from jax.experimental.pallas import tpu_sc as plsc

N_CORES = 2
N_SUB = 16
NW = N_CORES * N_SUB
CH = 50


def _dis(dg_ref):
    return lax.rsqrt(dg_ref[:, 0:1] + dg_ref[:, 1:2])


def _tc_mm(x, W1):
    N, F = x.shape
    BN = 2000

    def body(x_ref, w_ref, o_ref):
        o_ref[...] = jnp.dot(x_ref[...], w_ref[...],
                             preferred_element_type=jnp.float32,
                             precision=lax.Precision.HIGHEST)

    return pl.pallas_call(
        body,
        grid=(N // BN,),
        in_specs=[pl.BlockSpec((BN, F), lambda i: (i, 0)),
                  pl.BlockSpec((F, F), lambda i: (0, 0))],
        out_specs=pl.BlockSpec((BN, F), lambda i: (i, 0)),
        out_shape=jax.ShapeDtypeStruct((N, F), jnp.float32),
    )(x, W1)


def _tc_scale(h, degt):
    N, F = h.shape
    BN = 2000

    def body(h_ref, dg_ref, o_ref):
        o_ref[...] = h_ref[...] * _dis(dg_ref)

    return pl.pallas_call(
        body,
        grid=(N // BN,),
        in_specs=[pl.BlockSpec((BN, F), lambda i: (i, 0)),
                  pl.BlockSpec((BN, 2), lambda i: (i, 0))],
        out_specs=pl.BlockSpec((BN, F), lambda i: (i, 0)),
        out_shape=jax.ShapeDtypeStruct((N, F), jnp.float32),
    )(h, degt)


def _tc_relu(pa, pb, degt, b1):
    N, F = pa.shape
    BN = 2000

    def body(pa_ref, pb_ref, dg_ref, b_ref, o_ref):
        dis = _dis(dg_ref)
        agg = dis * (pa_ref[...] + pb_ref[...]) + b_ref[...]
        o_ref[...] = dis * jnp.maximum(agg, 0.0)

    return pl.pallas_call(
        body,
        grid=(N // BN,),
        in_specs=[pl.BlockSpec((BN, F), lambda i: (i, 0)),
                  pl.BlockSpec((BN, F), lambda i: (i, 0)),
                  pl.BlockSpec((BN, 2), lambda i: (i, 0)),
                  pl.BlockSpec((1, F), lambda i: (0, 0))],
        out_specs=pl.BlockSpec((BN, F), lambda i: (i, 0)),
        out_shape=jax.ShapeDtypeStruct((N, F), jnp.float32),
    )(pa, pb, degt, b1)


def _tc_out(pa, pb, degt, W2, b2):
    N, F = pa.shape
    F2 = W2.shape[1]
    BN = 2000

    def body(pa_ref, pb_ref, dg_ref, w_ref, b_ref, o_ref):
        dis = _dis(dg_ref)
        agg = dis * (pa_ref[...] + pb_ref[...])
        z = jnp.dot(agg, w_ref[...], preferred_element_type=jnp.float32,
                    precision=lax.Precision.HIGHEST) + b_ref[...]
        m = jnp.max(z, axis=1, keepdims=True)
        ez = jnp.exp(z - m)
        lse = jnp.log(jnp.sum(ez, axis=1, keepdims=True))
        o_ref[...] = (z - m) - lse

    return pl.pallas_call(
        body,
        grid=(N // BN,),
        in_specs=[pl.BlockSpec((BN, F), lambda i: (i, 0)),
                  pl.BlockSpec((BN, F), lambda i: (i, 0)),
                  pl.BlockSpec((BN, 2), lambda i: (i, 0)),
                  pl.BlockSpec((F, F2), lambda i: (0, 0)),
                  pl.BlockSpec((1, F2), lambda i: (0, 0))],
        out_specs=pl.BlockSpec((BN, F2), lambda i: (i, 0)),
        out_shape=jax.ShapeDtypeStruct((N, F2), jnp.float32),
    )(pa, pb, degt, W2, b2)


def _sc_degree(dst3, ones_np, zeros_np):
    NP = ones_np.shape[0]
    NCH, ch = dst3.shape[1], dst3.shape[2]
    RPS = NP // N_SUB
    NBD = 5
    mesh = plsc.VectorSubcoreMesh(core_axis_name="c", subcore_axis_name="s")

    @functools.partial(
        pl.kernel,
        out_type=[jax.ShapeDtypeStruct((NP,), jnp.float32),
                  jax.ShapeDtypeStruct((NP,), jnp.float32)],
        mesh=mesh,
        scratch_types=[
            pltpu.VMEM((NCH, ch), jnp.int32),
            pltpu.VMEM((ch,), jnp.float32),
            [pltpu.SemaphoreType.DMA for _ in range(NBD)],
            pltpu.VMEM_SHARED((NP,), jnp.float32),
        ],
    )
    def deg(dst_hbm, ones_hbm, z_hbm, out0_hbm, out1_hbm, dst_v, ones_v,
            dsem, acc):
        c = lax.axis_index("c")
        sub = lax.axis_index("s")
        wid = c * N_SUB + sub
        rs = pl.ds(sub * RPS, RPS)

        @pl.when(c == 0)
        def _():
            pltpu.sync_copy(ones_hbm.at[rs], acc.at[rs])

        @pl.when(c != 0)
        def _():
            pltpu.sync_copy(z_hbm.at[rs], acc.at[rs])

        pltpu.sync_copy(ones_hbm.at[pl.ds(0, ch)], ones_v)
        pltpu.sync_copy(dst_hbm.at[wid], dst_v)
        plsc.subcore_barrier()

        for b in range(NBD):
            pltpu.async_copy(ones_v, acc.at[dst_v.at[b]], dsem[b], add=True)

        @pl.loop(0, NCH // NBD - 1)
        def _(g):
            j0 = g * NBD
            for b in range(NBD):
                pltpu.make_async_copy(ones_v, acc.at[dst_v.at[j0 + b]],
                                      dsem[b]).wait()
                pltpu.async_copy(ones_v, acc.at[dst_v.at[j0 + NBD + b]],
                                 dsem[b], add=True)

        for b in range(NBD):
            pltpu.make_async_copy(ones_v, acc.at[dst_v.at[NCH - NBD + b]],
                                  dsem[b]).wait()

        plsc.subcore_barrier()

        @pl.when(c == 0)
        def _():
            pltpu.sync_copy(acc.at[rs], out0_hbm.at[rs])

        @pl.when(c != 0)
        def _():
            pltpu.sync_copy(acc.at[rs], out1_hbm.at[rs])

    return deg(dst3, ones_np, zeros_np)


def _sc_agg(s, src3, dst3, zeros_nf):
    N, F = s.shape
    nseg, seg, ch = src3.shape[1], src3.shape[2], src3.shape[3]
    CHR = 200
    NCHR = N // CHR
    mesh = plsc.VectorSubcoreMesh(core_axis_name="c", subcore_axis_name="s")

    NB = 5
    SEG = seg
    NSEG = nseg
    NGS = SEG // NB

    @functools.partial(
        pl.kernel,
        out_type=jax.ShapeDtypeStruct((N_CORES, N, F), jnp.float32),
        mesh=mesh,
        scratch_types=[
            pltpu.VMEM((SEG, ch), jnp.int32),
            pltpu.VMEM((SEG, ch), jnp.int32),
            [pltpu.VMEM((ch, F), jnp.float32) for _ in range(NB)],
            [pltpu.SemaphoreType.DMA for _ in range(NB)],
            [pltpu.SemaphoreType.DMA for _ in range(NB)],
            pltpu.VMEM_SHARED((N, F), jnp.float32),
        ],
    )
    def agg(s_hbm, src_hbm, dst_hbm, z_hbm, out_hbm, src_v, dst_v, rows,
            gsem, ssem, acc):
        c = lax.axis_index("c")
        sub = lax.axis_index("s")
        wid = c * N_SUB + sub

        @pl.loop(sub, NCHR, step=N_SUB)
        def _(k):
            rs = pl.ds(k * CHR, CHR)

            @pl.when(c == 0)
            def _():
                pltpu.sync_copy(s_hbm.at[rs], acc.at[rs])

            @pl.when(c != 0)
            def _():
                pltpu.sync_copy(z_hbm.at[rs], acc.at[rs])

        plsc.subcore_barrier()

        @pl.loop(0, NSEG)
        def _(sg):
            pltpu.sync_copy(src_hbm.at[wid, sg], src_v)
            pltpu.sync_copy(dst_hbm.at[wid, sg], dst_v)

            for b in range(NB):
                pltpu.async_copy(s_hbm.at[src_v.at[b]], rows[b], gsem[b])

            @pl.loop(0, NGS)
            def _(g):
                j0 = g * NB
                for b in range(NB):
                    pltpu.make_async_copy(s_hbm.at[src_v.at[j0 + b]], rows[b],
                                          gsem[b]).wait()
                    pltpu.async_copy(rows[b], acc.at[dst_v.at[j0 + b]],
                                     ssem[b], add=True)

                @pl.when(g < NGS - 1)
                def _():
                    for b in range(NB):
                        pltpu.make_async_copy(rows[b],
                                              acc.at[dst_v.at[j0 + b]],
                                              ssem[b]).wait()
                        pltpu.async_copy(s_hbm.at[src_v.at[j0 + NB + b]],
                                         rows[b], gsem[b])

            for b in range(NB):
                pltpu.make_async_copy(rows[b], acc.at[dst_v.at[SEG - NB + b]],
                                      ssem[b]).wait()

        plsc.subcore_barrier()

        @pl.loop(sub, NCHR, step=N_SUB)
        def _(k):
            rs = pl.ds(k * CHR, CHR)
            pltpu.sync_copy(acc.at[rs], out_hbm.at[c, rs])

    return agg(s, src3, dst3, zeros_nf)


def kernel(x, edge_index, W1, b1, W2, b2):
    N, F1 = x.shape
    F2 = W2.shape[1]
    E = edge_index.shape[1]

    ei = edge_index.astype(jnp.int32)
    NCH = E // (NW * CH)
    SEGC = 40
    src4 = ei[0].reshape(NW, NCH // SEGC, SEGC, CH)
    dst4 = ei[1].reshape(NW, NCH // SEGC, SEGC, CH)
    dst3 = ei[1].reshape(NW, NCH, CH)

    NP = -(-N // (16 * N_SUB)) * (16 * N_SUB)
    ones_np = jnp.ones((NP,), jnp.float32)
    zeros_np = jnp.zeros((NP,), jnp.float32)

    deg0, deg1 = _sc_degree(dst3, ones_np, zeros_np)
    degt = jnp.stack([deg0, deg1], axis=1)[:N]

    zeros_nf = jnp.zeros((N, F1), jnp.float32)
    h1 = _tc_mm(x, W1)
    s1 = _tc_scale(h1, degt)
    p1 = _sc_agg(s1, src4, dst4, zeros_nf)
    y = _tc_relu(p1[0], p1[1], degt, b1.reshape(1, F1))
    p2 = _sc_agg(y, src4, dst4, zeros_nf)
    return _tc_out(p2[0], p2[1], degt, W2, b2.reshape(1, F2))


# speedup vs baseline: 1.0153x; 1.0153x over previous
import functools

import jax
import jax.numpy as jnp
from jax import lax
from jax.experimental import pallas as pl
from jax.experimental.pallas import tpu as pltpu
from jax.experimental.pallas import tpu_sc as plsc

N_CORES = 2
N_SUB = 16
NW = N_CORES * N_SUB
CH = 40


def _dis(dg_ref):
    return lax.rsqrt(dg_ref[:, 0:1] + dg_ref[:, 1:2])


def _tc_mm(x, W1):
    N, F = x.shape
    BN = 2000

    def body(x_ref, w_ref, o_ref):
        o_ref[...] = jnp.dot(x_ref[...], w_ref[...],
                             preferred_element_type=jnp.float32,
                             precision=lax.Precision.HIGHEST)

    return pl.pallas_call(
        body,
        grid=(N // BN,),
        in_specs=[pl.BlockSpec((BN, F), lambda i: (i, 0)),
                  pl.BlockSpec((F, F), lambda i: (0, 0))],
        out_specs=pl.BlockSpec((BN, F), lambda i: (i, 0)),
        out_shape=jax.ShapeDtypeStruct((N, F), jnp.float32),
    )(x, W1)


def _tc_scale(h, degt):
    N, F = h.shape
    BN = 2000

    def body(h_ref, dg_ref, o_ref):
        o_ref[...] = h_ref[...] * _dis(dg_ref)

    return pl.pallas_call(
        body,
        grid=(N // BN,),
        in_specs=[pl.BlockSpec((BN, F), lambda i: (i, 0)),
                  pl.BlockSpec((BN, 2), lambda i: (i, 0))],
        out_specs=pl.BlockSpec((BN, F), lambda i: (i, 0)),
        out_shape=jax.ShapeDtypeStruct((N, F), jnp.float32),
    )(h, degt)


def _tc_relu(pa, pb, degt, b1):
    N, F = pa.shape
    BN = 2000

    def body(pa_ref, pb_ref, dg_ref, b_ref, o_ref):
        dis = _dis(dg_ref)
        agg = dis * (pa_ref[...] + pb_ref[...]) + b_ref[...]
        o_ref[...] = dis * jnp.maximum(agg, 0.0)

    return pl.pallas_call(
        body,
        grid=(N // BN,),
        in_specs=[pl.BlockSpec((BN, F), lambda i: (i, 0)),
                  pl.BlockSpec((BN, F), lambda i: (i, 0)),
                  pl.BlockSpec((BN, 2), lambda i: (i, 0)),
                  pl.BlockSpec((1, F), lambda i: (0, 0))],
        out_specs=pl.BlockSpec((BN, F), lambda i: (i, 0)),
        out_shape=jax.ShapeDtypeStruct((N, F), jnp.float32),
    )(pa, pb, degt, b1)


def _tc_out(pa, pb, degt, W2, b2):
    N, F = pa.shape
    F2 = W2.shape[1]
    BN = 2000

    def body(pa_ref, pb_ref, dg_ref, w_ref, b_ref, o_ref):
        dis = _dis(dg_ref)
        agg = dis * (pa_ref[...] + pb_ref[...])
        z = jnp.dot(agg, w_ref[...], preferred_element_type=jnp.float32,
                    precision=lax.Precision.HIGHEST) + b_ref[...]
        m = jnp.max(z, axis=1, keepdims=True)
        ez = jnp.exp(z - m)
        lse = jnp.log(jnp.sum(ez, axis=1, keepdims=True))
        o_ref[...] = (z - m) - lse

    return pl.pallas_call(
        body,
        grid=(N // BN,),
        in_specs=[pl.BlockSpec((BN, F), lambda i: (i, 0)),
                  pl.BlockSpec((BN, F), lambda i: (i, 0)),
                  pl.BlockSpec((BN, 2), lambda i: (i, 0)),
                  pl.BlockSpec((F, F2), lambda i: (0, 0)),
                  pl.BlockSpec((1, F2), lambda i: (0, 0))],
        out_specs=pl.BlockSpec((BN, F2), lambda i: (i, 0)),
        out_shape=jax.ShapeDtypeStruct((N, F2), jnp.float32),
    )(pa, pb, degt, W2, b2)


def _sc_degree(dst3, ones_np, zeros_np):
    NP = ones_np.shape[0]
    NCH, ch = dst3.shape[1], dst3.shape[2]
    RPS = NP // N_SUB
    NBD = 5
    mesh = plsc.VectorSubcoreMesh(core_axis_name="c", subcore_axis_name="s")

    @functools.partial(
        pl.kernel,
        out_type=[jax.ShapeDtypeStruct((NP,), jnp.float32),
                  jax.ShapeDtypeStruct((NP,), jnp.float32)],
        mesh=mesh,
        scratch_types=[
            pltpu.VMEM((NCH, ch), jnp.int32),
            pltpu.VMEM((ch,), jnp.float32),
            [pltpu.SemaphoreType.DMA for _ in range(NBD)],
            pltpu.VMEM_SHARED((NP,), jnp.float32),
        ],
    )
    def deg(dst_hbm, ones_hbm, z_hbm, out0_hbm, out1_hbm, dst_v, ones_v,
            dsem, acc):
        c = lax.axis_index("c")
        sub = lax.axis_index("s")
        wid = c * N_SUB + sub
        rs = pl.ds(sub * RPS, RPS)

        @pl.when(c == 0)
        def _():
            pltpu.sync_copy(ones_hbm.at[rs], acc.at[rs])

        @pl.when(c != 0)
        def _():
            pltpu.sync_copy(z_hbm.at[rs], acc.at[rs])

        pltpu.sync_copy(ones_hbm.at[pl.ds(0, ch)], ones_v)
        pltpu.sync_copy(dst_hbm.at[wid], dst_v)
        plsc.subcore_barrier()

        for b in range(NBD):
            pltpu.async_copy(ones_v, acc.at[dst_v.at[b]], dsem[b], add=True)

        @pl.loop(0, NCH // NBD - 1)
        def _(g):
            j0 = g * NBD
            for b in range(NBD):
                pltpu.make_async_copy(ones_v, acc.at[dst_v.at[j0 + b]],
                                      dsem[b]).wait()
                pltpu.async_copy(ones_v, acc.at[dst_v.at[j0 + NBD + b]],
                                 dsem[b], add=True)

        for b in range(NBD):
            pltpu.make_async_copy(ones_v, acc.at[dst_v.at[NCH - NBD + b]],
                                  dsem[b]).wait()

        plsc.subcore_barrier()

        @pl.when(c == 0)
        def _():
            pltpu.sync_copy(acc.at[rs], out0_hbm.at[rs])

        @pl.when(c != 0)
        def _():
            pltpu.sync_copy(acc.at[rs], out1_hbm.at[rs])

    return deg(dst3, ones_np, zeros_np)


def _sc_agg(s, src3, dst3, zeros_nf):
    N, F = s.shape
    nseg, seg, ch = src3.shape[1], src3.shape[2], src3.shape[3]
    CHR = 200
    NCHR = N // CHR
    mesh = plsc.VectorSubcoreMesh(core_axis_name="c", subcore_axis_name="s")

    NB = 5
    SEG = seg
    NSEG = nseg
    NGS = SEG // NB

    @functools.partial(
        pl.kernel,
        out_type=jax.ShapeDtypeStruct((N_CORES, N, F), jnp.float32),
        mesh=mesh,
        scratch_types=[
            pltpu.VMEM((SEG, ch), jnp.int32),
            pltpu.VMEM((SEG, ch), jnp.int32),
            [pltpu.VMEM((ch, F), jnp.float32) for _ in range(NB)],
            [pltpu.SemaphoreType.DMA for _ in range(NB)],
            [pltpu.SemaphoreType.DMA for _ in range(NB)],
            pltpu.VMEM_SHARED((N, F), jnp.float32),
        ],
    )
    def agg(s_hbm, src_hbm, dst_hbm, z_hbm, out_hbm, src_v, dst_v, rows,
            gsem, ssem, acc):
        c = lax.axis_index("c")
        sub = lax.axis_index("s")
        wid = c * N_SUB + sub

        @pl.loop(sub, NCHR, step=N_SUB)
        def _(k):
            rs = pl.ds(k * CHR, CHR)

            @pl.when(c == 0)
            def _():
                pltpu.sync_copy(s_hbm.at[rs], acc.at[rs])

            @pl.when(c != 0)
            def _():
                pltpu.sync_copy(z_hbm.at[rs], acc.at[rs])

        plsc.subcore_barrier()

        @pl.loop(0, NSEG)
        def _(sg):
            pltpu.sync_copy(src_hbm.at[wid, sg], src_v)
            pltpu.sync_copy(dst_hbm.at[wid, sg], dst_v)

            for b in range(NB):
                pltpu.async_copy(s_hbm.at[src_v.at[b]], rows[b], gsem[b])

            @pl.loop(0, NGS)
            def _(g):
                j0 = g * NB
                for b in range(NB):
                    pltpu.make_async_copy(s_hbm.at[src_v.at[j0 + b]], rows[b],
                                          gsem[b]).wait()
                    pltpu.async_copy(rows[b], acc.at[dst_v.at[j0 + b]],
                                     ssem[b], add=True)

                @pl.when(g < NGS - 1)
                def _():
                    for b in range(NB):
                        pltpu.make_async_copy(rows[b],
                                              acc.at[dst_v.at[j0 + b]],
                                              ssem[b]).wait()
                        pltpu.async_copy(s_hbm.at[src_v.at[j0 + NB + b]],
                                         rows[b], gsem[b])

            for b in range(NB):
                pltpu.make_async_copy(rows[b], acc.at[dst_v.at[SEG - NB + b]],
                                      ssem[b]).wait()

        plsc.subcore_barrier()

        @pl.loop(sub, NCHR, step=N_SUB)
        def _(k):
            rs = pl.ds(k * CHR, CHR)
            pltpu.sync_copy(acc.at[rs], out_hbm.at[c, rs])

    return agg(s, src3, dst3, zeros_nf)


def kernel(x, edge_index, W1, b1, W2, b2):
    N, F1 = x.shape
    F2 = W2.shape[1]
    E = edge_index.shape[1]

    ei = edge_index.astype(jnp.int32)
    NCH = E // (NW * CH)
    SEGC = 50
    src4 = ei[0].reshape(NW, NCH // SEGC, SEGC, CH)
    dst4 = ei[1].reshape(NW, NCH // SEGC, SEGC, CH)
    dst3 = ei[1].reshape(NW, NCH, CH)

    NP = -(-N // (16 * N_SUB)) * (16 * N_SUB)
    ones_np = jnp.ones((NP,), jnp.float32)
    zeros_np = jnp.zeros((NP,), jnp.float32)

    deg0, deg1 = _sc_degree(dst3, ones_np, zeros_np)
    degt = jnp.stack([deg0, deg1], axis=1)[:N]

    zeros_nf = jnp.zeros((N, F1), jnp.float32)
    h1 = _tc_mm(x, W1)
    s1 = _tc_scale(h1, degt)
    p1 = _sc_agg(s1, src4, dst4, zeros_nf)
    y = _tc_relu(p1[0], p1[1], degt, b1.reshape(1, F1))
    p2 = _sc_agg(y, src4, dst4, zeros_nf)
    return _tc_out(p2[0], p2[1], degt, W2, b2.reshape(1, F2))


# speedup vs baseline: 1.0697x; 1.0535x over previous
import functools

import jax
import jax.numpy as jnp
from jax import lax
from jax.experimental import pallas as pl
from jax.experimental.pallas import tpu as pltpu
from jax.experimental.pallas import tpu_sc as plsc

N_CORES = 2
N_SUB = 16
NW = N_CORES * N_SUB
CH = 40


def _dis(dg_ref):
    return lax.rsqrt(dg_ref[:, 0:1] + dg_ref[:, 1:2])


def _tc_lin1(x, W1, degt):
    N, F = x.shape
    BN = 2000

    def body(x_ref, w_ref, dg_ref, o_ref):
        h = jnp.dot(x_ref[...], w_ref[...], preferred_element_type=jnp.float32,
                    precision=lax.Precision.HIGHEST)
        o_ref[...] = h * _dis(dg_ref)

    return pl.pallas_call(
        body,
        grid=(N // BN,),
        in_specs=[pl.BlockSpec((BN, F), lambda i: (i, 0)),
                  pl.BlockSpec((F, F), lambda i: (0, 0)),
                  pl.BlockSpec((BN, 2), lambda i: (i, 0))],
        out_specs=pl.BlockSpec((BN, F), lambda i: (i, 0)),
        out_shape=jax.ShapeDtypeStruct((N, F), jnp.float32),
    )(x, W1, degt)


def _tc_relu(pa, pb, degt, b1):
    N, F = pa.shape
    BN = 2000

    def body(pa_ref, pb_ref, dg_ref, b_ref, o_ref):
        dis = _dis(dg_ref)
        agg = dis * (pa_ref[...] + pb_ref[...]) + b_ref[...]
        o_ref[...] = dis * jnp.maximum(agg, 0.0)

    return pl.pallas_call(
        body,
        grid=(N // BN,),
        in_specs=[pl.BlockSpec((BN, F), lambda i: (i, 0)),
                  pl.BlockSpec((BN, F), lambda i: (i, 0)),
                  pl.BlockSpec((BN, 2), lambda i: (i, 0)),
                  pl.BlockSpec((1, F), lambda i: (0, 0))],
        out_specs=pl.BlockSpec((BN, F), lambda i: (i, 0)),
        out_shape=jax.ShapeDtypeStruct((N, F), jnp.float32),
    )(pa, pb, degt, b1)


def _tc_out(pa, pb, degt, W2, b2):
    N, F = pa.shape
    F2 = W2.shape[1]
    BN = 2000

    def body(pa_ref, pb_ref, dg_ref, w_ref, b_ref, o_ref):
        dis = _dis(dg_ref)
        agg = dis * (pa_ref[...] + pb_ref[...])
        z = jnp.dot(agg, w_ref[...], preferred_element_type=jnp.float32,
                    precision=lax.Precision.HIGHEST) + b_ref[...]
        m = jnp.max(z, axis=1, keepdims=True)
        ez = jnp.exp(z - m)
        lse = jnp.log(jnp.sum(ez, axis=1, keepdims=True))
        o_ref[...] = (z - m) - lse

    return pl.pallas_call(
        body,
        grid=(N // BN,),
        in_specs=[pl.BlockSpec((BN, F), lambda i: (i, 0)),
                  pl.BlockSpec((BN, F), lambda i: (i, 0)),
                  pl.BlockSpec((BN, 2), lambda i: (i, 0)),
                  pl.BlockSpec((F, F2), lambda i: (0, 0)),
                  pl.BlockSpec((1, F2), lambda i: (0, 0))],
        out_specs=pl.BlockSpec((BN, F2), lambda i: (i, 0)),
        out_shape=jax.ShapeDtypeStruct((N, F2), jnp.float32),
    )(pa, pb, degt, W2, b2)


def _sc_degree(dst3, ones_np, zeros_np):
    NP = ones_np.shape[0]
    NCH, ch = dst3.shape[1], dst3.shape[2]
    RPS = NP // N_SUB
    NBD = 5
    mesh = plsc.VectorSubcoreMesh(core_axis_name="c", subcore_axis_name="s")

    @functools.partial(
        pl.kernel,
        out_type=[jax.ShapeDtypeStruct((NP,), jnp.float32),
                  jax.ShapeDtypeStruct((NP,), jnp.float32)],
        mesh=mesh,
        scratch_types=[
            pltpu.VMEM((NCH, ch), jnp.int32),
            pltpu.VMEM((ch,), jnp.float32),
            [pltpu.SemaphoreType.DMA for _ in range(NBD)],
            pltpu.VMEM_SHARED((NP,), jnp.float32),
        ],
    )
    def deg(dst_hbm, ones_hbm, z_hbm, out0_hbm, out1_hbm, dst_v, ones_v,
            dsem, acc):
        c = lax.axis_index("c")
        sub = lax.axis_index("s")
        wid = c * N_SUB + sub
        rs = pl.ds(sub * RPS, RPS)

        @pl.when(c == 0)
        def _():
            pltpu.sync_copy(ones_hbm.at[rs], acc.at[rs])

        @pl.when(c != 0)
        def _():
            pltpu.sync_copy(z_hbm.at[rs], acc.at[rs])

        pltpu.sync_copy(ones_hbm.at[pl.ds(0, ch)], ones_v)
        pltpu.sync_copy(dst_hbm.at[wid], dst_v)
        plsc.subcore_barrier()

        for b in range(NBD):
            pltpu.async_copy(ones_v, acc.at[dst_v.at[b]], dsem[b], add=True)

        @pl.loop(0, NCH // NBD - 1)
        def _(g):
            j0 = g * NBD
            for b in range(NBD):
                pltpu.make_async_copy(ones_v, acc.at[dst_v.at[j0 + b]],
                                      dsem[b]).wait()
                pltpu.async_copy(ones_v, acc.at[dst_v.at[j0 + NBD + b]],
                                 dsem[b], add=True)

        for b in range(NBD):
            pltpu.make_async_copy(ones_v, acc.at[dst_v.at[NCH - NBD + b]],
                                  dsem[b]).wait()

        plsc.subcore_barrier()

        @pl.when(c == 0)
        def _():
            pltpu.sync_copy(acc.at[rs], out0_hbm.at[rs])

        @pl.when(c != 0)
        def _():
            pltpu.sync_copy(acc.at[rs], out1_hbm.at[rs])

    return deg(dst3, ones_np, zeros_np)


def _sc_agg(s, src3, dst3, zeros_nf):
    N, F = s.shape
    nseg, seg, ch = src3.shape[1], src3.shape[2], src3.shape[3]
    CHR = 200
    NCHR = N // CHR
    mesh = plsc.VectorSubcoreMesh(core_axis_name="c", subcore_axis_name="s")

    NB = 5
    SEG = seg
    NSEG = nseg
    NGS = SEG // NB

    @functools.partial(
        pl.kernel,
        out_type=[jax.ShapeDtypeStruct((N, F), jnp.float32),
                  jax.ShapeDtypeStruct((N, F), jnp.float32)],
        mesh=mesh,
        scratch_types=[
            pltpu.VMEM((SEG, ch), jnp.int32),
            pltpu.VMEM((SEG, ch), jnp.int32),
            [pltpu.VMEM((ch, F), jnp.float32) for _ in range(NB)],
            [pltpu.SemaphoreType.DMA for _ in range(NB)],
            [pltpu.SemaphoreType.DMA for _ in range(NB)],
            pltpu.VMEM_SHARED((N, F), jnp.float32),
        ],
    )
    def agg(s_hbm, src_hbm, dst_hbm, z_hbm, out0_hbm, out1_hbm, src_v, dst_v,
            rows, gsem, ssem, acc):
        c = lax.axis_index("c")
        sub = lax.axis_index("s")
        wid = c * N_SUB + sub

        @pl.loop(sub, NCHR, step=N_SUB)
        def _(k):
            rs = pl.ds(k * CHR, CHR)

            @pl.when(c == 0)
            def _():
                pltpu.sync_copy(s_hbm.at[rs], acc.at[rs])

            @pl.when(c != 0)
            def _():
                pltpu.sync_copy(z_hbm.at[rs], acc.at[rs])

        plsc.subcore_barrier()

        @pl.loop(0, NSEG)
        def _(sg):
            pltpu.sync_copy(src_hbm.at[wid, sg], src_v)
            pltpu.sync_copy(dst_hbm.at[wid, sg], dst_v)

            for b in range(NB):
                pltpu.async_copy(s_hbm.at[src_v.at[b]], rows[b], gsem[b])

            @pl.loop(0, NGS)
            def _(g):
                j0 = g * NB
                for b in range(NB):
                    pltpu.make_async_copy(s_hbm.at[src_v.at[j0 + b]], rows[b],
                                          gsem[b]).wait()
                    pltpu.async_copy(rows[b], acc.at[dst_v.at[j0 + b]],
                                     ssem[b], add=True)

                @pl.when(g < NGS - 1)
                def _():
                    for b in range(NB):
                        pltpu.make_async_copy(rows[b],
                                              acc.at[dst_v.at[j0 + b]],
                                              ssem[b]).wait()
                        pltpu.async_copy(s_hbm.at[src_v.at[j0 + NB + b]],
                                         rows[b], gsem[b])

            for b in range(NB):
                pltpu.make_async_copy(rows[b], acc.at[dst_v.at[SEG - NB + b]],
                                      ssem[b]).wait()

        plsc.subcore_barrier()

        @pl.loop(sub, NCHR, step=N_SUB)
        def _(k):
            rs = pl.ds(k * CHR, CHR)

            @pl.when(c == 0)
            def _():
                pltpu.sync_copy(acc.at[rs], out0_hbm.at[rs])

            @pl.when(c != 0)
            def _():
                pltpu.sync_copy(acc.at[rs], out1_hbm.at[rs])

    return agg(s, src3, dst3, zeros_nf)


def kernel(x, edge_index, W1, b1, W2, b2):
    N, F1 = x.shape
    F2 = W2.shape[1]
    E = edge_index.shape[1]

    ei = edge_index.astype(jnp.int32)
    NCH = E // (NW * CH)
    SEGC = 50
    src4 = ei[0].reshape(NW, NCH // SEGC, SEGC, CH)
    dst4 = ei[1].reshape(NW, NCH // SEGC, SEGC, CH)
    dst3 = ei[1].reshape(NW, NCH, CH)

    NP = -(-N // (16 * N_SUB)) * (16 * N_SUB)
    ones_np = jnp.ones((NP,), jnp.float32)
    zeros_np = jnp.zeros((NP,), jnp.float32)

    deg0, deg1 = _sc_degree(dst3, ones_np, zeros_np)
    degt = jnp.stack([deg0, deg1], axis=1)[:N]

    zeros_nf = jnp.zeros((N, F1), jnp.float32)
    s1 = _tc_lin1(x, W1, degt)
    p1a, p1b = _sc_agg(s1, src4, dst4, zeros_nf)
    y = _tc_relu(p1a, p1b, degt, b1.reshape(1, F1))
    p2a, p2b = _sc_agg(y, src4, dst4, zeros_nf)
    return _tc_out(p2a, p2b, degt, W2, b2.reshape(1, F2))
